# Initial kernel scaffold; baseline (speedup 1.0000x reference)
import jax
import jax.numpy as jnp
from jax.experimental import pallas as pl


def kernel(x, edge_index, edge_attr, pos, batch, We1, be1, Wx1, bx1, Wh1, bh1, We2, be2, Wx2, bx2, Wh2, bh2, We3, be3, Wx3, bx3, Wh3, bh3, Wl, bl, Wl2, bl2):
    raise NotImplementedError("write your pallas kernel here")


# speedup vs baseline: 5.0716x; 5.0716x over previous
import functools

import jax
import jax.numpy as jnp
from jax import lax
from jax.experimental import pallas as pl
from jax.experimental.pallas import tpu as pltpu
from jax.experimental.pallas import tpu_sc as plsc

N = 50000
NG = 64
NPAD = 50048
SLAB = NPAD // 16
E = 800000
C = 3072
CR = C // 128
KCH = 9
EPW = C * KCH
EPAD = EPW * 32
HID = 128
TD = 8

_f32 = jnp.float32
_i32 = jnp.int32
_HI = lax.Precision.HIGHEST


_mesh = plsc.VectorSubcoreMesh(core_axis_name="c", subcore_axis_name="s")


@functools.partial(
    pl.kernel,
    out_type=jax.ShapeDtypeStruct((2, NPAD, TD), _f32),
    mesh=_mesh,
    compiler_params=pltpu.CompilerParams(needs_layout_passes=False,
                                         use_tc_tiling_on_sc=False),
    scratch_types=[
        pltpu.VMEM((CR, 128), _i32),
        pltpu.VMEM((CR, 128), _i32),
        pltpu.VMEM((C, 4), _f32),
        pltpu.VMEM((C, TD), _f32),
        pltpu.VMEM((C, TD), _f32),
        pltpu.VMEM((C, TD), _f32),
        pltpu.VMEM((16,), _f32),
        pltpu.VMEM_SHARED((NPAD, TD), _f32),
        pltpu.SemaphoreType.DMA,
    ],
)
def _sc_edge_pass(tab_hbm, src_hbm, dst_hbm, ea_hbm, par_hbm, zer_hbm,
                  out_hbm, src_v, dst_v, ea_v, ts_v, td_v, msg_v, par_v, agg,
                  sem):
    cid = lax.axis_index("c")
    sid = lax.axis_index("s")
    wid = cid * 16 + sid

    pltpu.sync_copy(zer_hbm, agg.at[pl.ds(sid * SLAB, SLAB)])
    pltpu.sync_copy(zer_hbm.at[pl.ds(0, C)], msg_v)
    pltpu.sync_copy(par_hbm, par_v)
    plsc.subcore_barrier()

    zero16 = jnp.zeros((16,), _i32)

    def bc(j):
        return plsc.load_gather(par_v, [zero16 + j])

    w_d, w_x, b_x = bc(1), bc(2), bc(3)
    wa0, wa1, wa2, wa3 = bc(4), bc(5), bc(6), bc(7)
    be = bc(8)

    iota16 = lax.iota(_i32, 16)
    c0, c1, c2, c3 = zero16, zero16 + 1, zero16 + 2, zero16 + 3
    c4 = zero16 + 4
    magic = jnp.full((16,), 0x5F3759DF, _i32)

    ebase = wid * EPW
    rbase = wid * (CR * KCH)
    for k in range(KCH):
        eoff = ebase + k * C
        roff = rbase + k * CR
        pltpu.sync_copy(src_hbm.at[pl.ds(roff, CR)], src_v)
        pltpu.sync_copy(dst_hbm.at[pl.ds(roff, CR)], dst_v)
        pltpu.sync_copy(ea_hbm.at[pl.ds(eoff, C)], ea_v)

        cps = []
        for j in range(CR):
            cps.append(pltpu.async_copy(
                tab_hbm.at[src_v.at[j]], ts_v.at[pl.ds(j * 128, 128)], sem))
            cps.append(pltpu.async_copy(
                tab_hbm.at[dst_v.at[j]], td_v.at[pl.ds(j * 128, 128)], sem))
        for cp in cps:
            cp.wait()

        def body(i, carry):
            rows = iota16 + i * 16
            a_s = plsc.load_gather(ts_v, [rows, c0])
            sx = plsc.load_gather(ts_v, [rows, c2])
            sy = plsc.load_gather(ts_v, [rows, c3])
            sz = plsc.load_gather(ts_v, [rows, c4])
            a_d = plsc.load_gather(td_v, [rows, c1])
            tx = plsc.load_gather(td_v, [rows, c2])
            ty = plsc.load_gather(td_v, [rows, c3])
            tz = plsc.load_gather(td_v, [rows, c4])
            ea0 = plsc.load_gather(ea_v, [rows, c0])
            ea1 = plsc.load_gather(ea_v, [rows, c1])
            ea2 = plsc.load_gather(ea_v, [rows, c2])
            ea3 = plsc.load_gather(ea_v, [rows, c3])

            ec = ea0 * wa0 + ea1 * wa1 + ea2 * wa2 + ea3 * wa3
            dx = sx - tx
            dy = sy - ty
            dz = sz - tz
            d2 = dx * dx + dy * dy + dz * dz
            yi = magic - (plsc.bitcast(d2, _i32) >> 1)
            y = plsc.bitcast(yi, _f32)
            hx = 0.5 * d2
            y = y * (1.5 - hx * y * y)
            y = y * (1.5 - hx * y * y)
            y = y * (1.5 - hx * y * y)
            dist = d2 * y

            m = a_s + a_d + ec + w_d * dist + be
            info = m * w_x + b_x
            plsc.store_scatter(msg_v, [rows, c0], m)
            plsc.store_scatter(msg_v, [rows, c1], dx * info)
            plsc.store_scatter(msg_v, [rows, c2], dy * info)
            plsc.store_scatter(msg_v, [rows, c3], dz * info)
            return carry

        lax.fori_loop(0, C // 16, body, 0)

        for j in range(CR):
            pltpu.sync_copy(msg_v.at[pl.ds(j * 128, 128)],
                            agg.at[src_v.at[j]], add=True)

    plsc.subcore_barrier()
    pltpu.sync_copy(agg.at[pl.ds(sid * SLAB, SLAB)],
                    out_hbm.at[cid, pl.ds(sid * SLAB, SLAB)])


def _prep_body(x_ref, pos_ref, we_ref, tab_ref):
    xb = x_ref[...]
    we = we_ref[...]
    ai = jnp.dot(xb, we[0:11, :], preferred_element_type=_f32, precision=_HI)
    aj = jnp.dot(xb, we[11:22, :], preferred_element_type=_f32, precision=_HI)
    p = pos_ref[...]
    z = jnp.zeros((xb.shape[0], 3), _f32)
    tab_ref[...] = jnp.concatenate([ai, aj, p, z], axis=1)


def _tc_prep(xp, posp, We1):
    return pl.pallas_call(
        _prep_body,
        grid=(NPAD // SLAB,),
        in_specs=[
            pl.BlockSpec((SLAB, 11), lambda i: (i, 0)),
            pl.BlockSpec((SLAB, 3), lambda i: (i, 0)),
            pl.BlockSpec((27, 1), lambda i: (0, 0)),
        ],
        out_specs=pl.BlockSpec((SLAB, TD), lambda i: (i, 0)),
        out_shape=jax.ShapeDtypeStruct((NPAD, TD), _f32),
    )(xp, posp, We1)


def _layer_body(F, x_ref, tabp_ref, agg_ref, wh_ref, bh_ref, wen_ref,
                xo_ref, tab_ref):
    xb = x_ref[...]
    aggm = agg_ref[0, :, 0:1] + agg_ref[1, :, 0:1]
    aggz = agg_ref[0, :, 1:4] + agg_ref[1, :, 1:4]
    h = jnp.dot(xb, wh_ref[0:F, :], preferred_element_type=_f32)
    h = h + aggm * wh_ref[F:F + 1, :] + bh_ref[...]
    xo = jnp.maximum(h, 0.0)
    xo_ref[...] = xo
    cn = tabp_ref[:, 2:5] + aggz * (1.0 / N)
    ai = jnp.dot(xo, wen_ref[0:HID, :], preferred_element_type=_f32,
                 precision=_HI)
    aj = jnp.dot(xo, wen_ref[HID:2 * HID, :], preferred_element_type=_f32,
                 precision=_HI)
    z = jnp.zeros((xb.shape[0], 3), _f32)
    tab_ref[...] = jnp.concatenate([ai, aj, cn, z], axis=1)


def _tc_layer(F, xp, tab_prev, aggpair, Wh, bh, Wen):
    return pl.pallas_call(
        functools.partial(_layer_body, F),
        grid=(NPAD // SLAB,),
        in_specs=[
            pl.BlockSpec((SLAB, F), lambda i: (i, 0)),
            pl.BlockSpec((SLAB, TD), lambda i: (i, 0)),
            pl.BlockSpec((2, SLAB, TD), lambda i: (0, i, 0)),
            pl.BlockSpec((F + 1, HID), lambda i: (0, 0)),
            pl.BlockSpec((1, HID), lambda i: (0, 0)),
            pl.BlockSpec((2 * HID + 5, 1), lambda i: (0, 0)),
        ],
        out_specs=[
            pl.BlockSpec((SLAB, HID), lambda i: (i, 0)),
            pl.BlockSpec((SLAB, TD), lambda i: (i, 0)),
        ],
        out_shape=[
            jax.ShapeDtypeStruct((NPAD, HID), _f32),
            jax.ShapeDtypeStruct((NPAD, TD), _f32),
        ],
    )(xp, tab_prev, aggpair, Wh, bh, Wen)


def _final_body(x_ref, agg_ref, wh_ref, bh_ref, b_ref, wl_ref, bl_ref,
                wl2_ref, bl2_ref, out_ref, acc_ref):
    i = pl.program_id(0)
    xb = x_ref[...]
    aggm = agg_ref[0, :, 0:1] + agg_ref[1, :, 0:1]
    h = jnp.dot(xb, wh_ref[0:HID, :], preferred_element_type=_f32)
    h = h + aggm * wh_ref[HID:HID + 1, :] + bh_ref[...]
    x3 = jnp.maximum(h, 0.0)
    bblk = b_ref[0, 0, :]
    oh = (bblk[:, None] == lax.broadcasted_iota(_i32, (SLAB, NG), 1))
    part = lax.dot_general(oh.astype(_f32), x3, (((0,), (0,)), ((), ())),
                           preferred_element_type=_f32, precision=_HI)

    @pl.when(i == 0)
    def _():
        acc_ref[...] = part

    @pl.when(i != 0)
    def _():
        acc_ref[...] = acc_ref[...] + part

    @pl.when(i == (NPAD // SLAB) - 1)
    def _():
        pooled = acc_ref[...]
        hh = jnp.maximum(
            jnp.dot(pooled, wl_ref[...], preferred_element_type=_f32,
                    precision=_HI) + bl_ref[...], 0.0)
        out_ref[...] = (jnp.dot(hh, wl2_ref[...], preferred_element_type=_f32,
                                precision=_HI) + bl2_ref[...])


def _tc_final(x2, aggpair, Wh3, bh3, batch3d, Wl, bl, Wl2, bl2):
    return pl.pallas_call(
        _final_body,
        grid=(NPAD // SLAB,),
        in_specs=[
            pl.BlockSpec((SLAB, HID), lambda i: (i, 0)),
            pl.BlockSpec((2, SLAB, TD), lambda i: (0, i, 0)),
            pl.BlockSpec((HID + 1, HID), lambda i: (0, 0)),
            pl.BlockSpec((1, HID), lambda i: (0, 0)),
            pl.BlockSpec((1, 1, SLAB), lambda i: (i, 0, 0)),
            pl.BlockSpec((HID, NG), lambda i: (0, 0)),
            pl.BlockSpec((1, NG), lambda i: (0, 0)),
            pl.BlockSpec((NG, 1), lambda i: (0, 0)),
            pl.BlockSpec((1, 1), lambda i: (0, 0)),
        ],
        out_specs=pl.BlockSpec((NG, 1), lambda i: (0, 0)),
        out_shape=jax.ShapeDtypeStruct((NG, 1), _f32),
        scratch_shapes=[pltpu.VMEM((NG, HID), _f32)],
    )(x2, aggpair, Wh3, bh3, batch3d, Wl, bl, Wl2, bl2)


def _params16(We, Wx, bx, be, F):
    return jnp.concatenate([
        jnp.zeros((1,), _f32),
        We[2 * F + 4, :], Wx[0, :], bx,
        We[2 * F + 0, :], We[2 * F + 1, :], We[2 * F + 2, :], We[2 * F + 3, :],
        be, jnp.zeros((7,), _f32),
    ])


def kernel(x, edge_index, edge_attr, pos, batch,
           We1, be1, Wx1, bx1, Wh1, bh1,
           We2, be2, Wx2, bx2, Wh2, bh2,
           We3, be3, Wx3, bx3, Wh3, bh3,
           Wl, bl, Wl2, bl2):
    src = edge_index[0].astype(_i32)
    dst = edge_index[1].astype(_i32)
    padi = jnp.full((EPAD - E,), N, _i32)
    srcp = jnp.concatenate([src, padi]).reshape(EPAD // 128, 128)
    dstp = jnp.concatenate([dst, padi]).reshape(EPAD // 128, 128)
    eap = jnp.concatenate([edge_attr,
                           jnp.zeros((EPAD - E, 4), _f32)], axis=0)
    xp = jnp.pad(x, ((0, NPAD - N), (0, 0)))
    posp = jnp.pad(pos, ((0, NPAD - N), (0, 0)))
    batchp = jnp.concatenate(
        [batch.astype(_i32), jnp.full((NPAD - N,), NG, _i32)]
    ).reshape(NPAD // SLAB, 1, SLAB)
    zer = jnp.zeros((SLAB, TD), _f32)

    par1 = _params16(We1, Wx1, bx1, be1, 11)
    par2 = _params16(We2, Wx2, bx2, be2, HID)
    par3 = _params16(We3, Wx3, bx3, be3, HID)

    tab1 = _tc_prep(xp, posp, We1)
    agg1 = _sc_edge_pass(tab1, srcp, dstp, eap, par1, zer)
    x1, tab2 = _tc_layer(11, xp, tab1, agg1, Wh1, bh1.reshape(1, HID), We2)
    agg2 = _sc_edge_pass(tab2, srcp, dstp, eap, par2, zer)
    x2, tab3 = _tc_layer(HID, x1, tab2, agg2, Wh2, bh2.reshape(1, HID), We3)
    agg3 = _sc_edge_pass(tab3, srcp, dstp, eap, par3, zer)
    out = _tc_final(x2, agg3, Wh3, bh3.reshape(1, HID), batchp,
                    Wl, bl.reshape(1, NG), Wl2, bl2.reshape(1, 1))
    return out


# speedup vs baseline: 10.3907x; 2.0488x over previous
import functools

import jax
import jax.numpy as jnp
from jax import lax
from jax.experimental import pallas as pl
from jax.experimental.pallas import tpu as pltpu
from jax.experimental.pallas import tpu_sc as plsc

N = 50000
NG = 64
NPAD = 50048
SLAB = NPAD // 16
E = 800000
EROWS = E // 128
CR = 10
C = CR * 128
NCHUNK = EROWS // CR
KCH = 20
HID = 128
TD = 8

_f32 = jnp.float32
_i32 = jnp.int32
_HI = lax.Precision.HIGHEST


_mesh = plsc.VectorSubcoreMesh(core_axis_name="c", subcore_axis_name="s")


@functools.partial(
    pl.kernel,
    out_type=jax.ShapeDtypeStruct((2, NPAD, TD), _f32),
    mesh=_mesh,
    compiler_params=pltpu.CompilerParams(needs_layout_passes=False,
                                         use_tc_tiling_on_sc=False),
    scratch_types=[
        [pltpu.VMEM((CR, 128), _i32)] * 2,
        pltpu.VMEM((CR, 128), _i32),
        [pltpu.VMEM((C, 4), _f32)] * 2,
        [pltpu.VMEM((C, TD), _f32)] * 2,
        [pltpu.VMEM((C, TD), _f32)] * 2,
        [pltpu.VMEM((C, TD), _f32)] * 2,
        pltpu.VMEM((16,), _f32),
        pltpu.VMEM_SHARED((NPAD, TD), _f32),
        pltpu.SemaphoreType.DMA,
        pltpu.SemaphoreType.DMA,
    ],
)
def _sc_edge_pass(tab_hbm, src_hbm, dst_hbm, ea_hbm, par_hbm, zer_hbm,
                  out_hbm, src_v, dst_v, ea_v, ts_v, td_v, msg_v, par_v, agg,
                  gsem, ssem):
    cid = lax.axis_index("c")
    sid = lax.axis_index("s")
    wid = cid * 16 + sid

    pltpu.sync_copy(zer_hbm.at[pl.ds(0, SLAB)], agg.at[pl.ds(sid * SLAB, SLAB)])
    pltpu.sync_copy(zer_hbm.at[pl.ds(0, C)], msg_v[0])
    pltpu.sync_copy(zer_hbm.at[pl.ds(0, C)], msg_v[1])
    pltpu.sync_copy(par_hbm, par_v)
    plsc.subcore_barrier()

    zero16 = jnp.zeros((16,), _i32)

    def bc(j):
        return plsc.load_gather(par_v, [zero16 + j])

    w_d, w_x, b_x = bc(1), bc(2), bc(3)
    wa0, wa1, wa2, wa3 = bc(4), bc(5), bc(6), bc(7)
    be = bc(8)

    iota16 = lax.iota(_i32, 16)
    c0, c1, c2, c3 = zero16, zero16 + 1, zero16 + 2, zero16 + 3
    c4 = zero16 + 4
    magic = jnp.full((16,), 0x5F3759DF, _i32)

    def chunk_of(k):
        if k == KCH - 1:
            ch = jnp.where(wid + 32 * k < NCHUNK, wid + 32 * k, wid)
            vmask = jnp.where(zero16 + (wid + 32 * k) < NCHUNK, 1.0, 0.0)
            return ch, vmask
        return wid + 32 * k, None

    def fire_chunk(k, buf):
        ch, _ = chunk_of(k)
        pltpu.sync_copy(src_hbm.at[pl.ds(ch * CR, CR)], src_v[buf])
        pltpu.sync_copy(dst_hbm.at[pl.ds(ch * CR, CR)], dst_v)
        pltpu.sync_copy(ea_hbm.at[pl.ds(ch * C, C)], ea_v[buf])
        cps = []
        for j in range(CR):
            cps.append(pltpu.async_copy(
                tab_hbm.at[src_v[buf].at[j]],
                ts_v[buf].at[pl.ds(j * 128, 128)], gsem))
            cps.append(pltpu.async_copy(
                tab_hbm.at[dst_v.at[j]],
                td_v[buf].at[pl.ds(j * 128, 128)], gsem))
        return cps

    def compute_chunk(k, buf):
        _, vmask = chunk_of(k)
        msg = msg_v[buf]
        tsb = ts_v[buf]
        tdb = td_v[buf]
        eab = ea_v[buf]

        def body(i, carry):
            rows = iota16 + i * 16
            a_s = plsc.load_gather(tsb, [rows, c0])
            sx = plsc.load_gather(tsb, [rows, c2])
            sy = plsc.load_gather(tsb, [rows, c3])
            sz = plsc.load_gather(tsb, [rows, c4])
            a_d = plsc.load_gather(tdb, [rows, c1])
            tx = plsc.load_gather(tdb, [rows, c2])
            ty = plsc.load_gather(tdb, [rows, c3])
            tz = plsc.load_gather(tdb, [rows, c4])
            ea0 = plsc.load_gather(eab, [rows, c0])
            ea1 = plsc.load_gather(eab, [rows, c1])
            ea2 = plsc.load_gather(eab, [rows, c2])
            ea3 = plsc.load_gather(eab, [rows, c3])

            ec = ea0 * wa0 + ea1 * wa1 + ea2 * wa2 + ea3 * wa3
            dx = sx - tx
            dy = sy - ty
            dz = sz - tz
            d2 = dx * dx + dy * dy + dz * dz
            yi = magic - (plsc.bitcast(d2, _i32) >> 1)
            y = plsc.bitcast(yi, _f32)
            hx = 0.5 * d2
            y = y * (1.5 - hx * y * y)
            y = y * (1.5 - hx * y * y)
            y = y * (1.5 - hx * y * y)
            dist = d2 * y

            m = a_s + a_d + ec + w_d * dist + be
            info = m * w_x + b_x
            zx = dx * info
            zy = dy * info
            zz = dz * info
            if vmask is not None:
                m = m * vmask
                zx = zx * vmask
                zy = zy * vmask
                zz = zz * vmask
            plsc.store_scatter(msg, [rows, c0], m)
            plsc.store_scatter(msg, [rows, c1], zx)
            plsc.store_scatter(msg, [rows, c2], zy)
            plsc.store_scatter(msg, [rows, c3], zz)
            return carry

        lax.fori_loop(0, C // 16, body, 0)

        return [pltpu.async_copy(msg.at[pl.ds(j * 128, 128)],
                                 agg.at[src_v[buf].at[j]], ssem, add=True)
                for j in range(CR)]

    gath = {0: fire_chunk(0, 0)}
    scat = {}
    for k in range(KCH):
        buf = k % 2
        for cp in gath.pop(k):
            cp.wait()
        if k + 1 < KCH:
            if k - 1 in scat:
                for cp in scat.pop(k - 1):
                    cp.wait()
            gath[k + 1] = fire_chunk(k + 1, (k + 1) % 2)
        scat[k] = compute_chunk(k, buf)

    for _, scs in scat.items():
        for cp in scs:
            cp.wait()

    plsc.subcore_barrier()
    pltpu.sync_copy(agg.at[pl.ds(sid * SLAB, SLAB)],
                    out_hbm.at[cid, pl.ds(sid * SLAB, SLAB)])


def _prep_body(x_ref, pos_ref, we_ref, tab_ref):
    xb = x_ref[...]
    we = we_ref[...]
    ai = jnp.dot(xb, we[0:11, :], preferred_element_type=_f32, precision=_HI)
    aj = jnp.dot(xb, we[11:22, :], preferred_element_type=_f32, precision=_HI)
    p = pos_ref[...]
    z = jnp.zeros((xb.shape[0], 3), _f32)
    tab_ref[...] = jnp.concatenate([ai, aj, p, z], axis=1)


def _tc_prep(xp, posp, We1):
    return pl.pallas_call(
        _prep_body,
        grid=(NPAD // SLAB,),
        in_specs=[
            pl.BlockSpec((SLAB, 11), lambda i: (i, 0)),
            pl.BlockSpec((SLAB, 3), lambda i: (i, 0)),
            pl.BlockSpec((27, 1), lambda i: (0, 0)),
        ],
        out_specs=pl.BlockSpec((SLAB, TD), lambda i: (i, 0)),
        out_shape=jax.ShapeDtypeStruct((NPAD, TD), _f32),
    )(xp, posp, We1)


def _layer_body(F, x_ref, tabp_ref, agg_ref, wh_ref, bh_ref, wen_ref,
                xo_ref, tab_ref):
    xb = x_ref[...]
    aggm = agg_ref[0, :, 0:1] + agg_ref[1, :, 0:1]
    aggz = agg_ref[0, :, 1:4] + agg_ref[1, :, 1:4]
    h = jnp.dot(xb, wh_ref[0:F, :], preferred_element_type=_f32)
    h = h + aggm * wh_ref[F:F + 1, :] + bh_ref[...]
    xo = jnp.maximum(h, 0.0)
    xo_ref[...] = xo
    cn = tabp_ref[:, 2:5] + aggz * (1.0 / N)
    ai = jnp.dot(xo, wen_ref[0:HID, :], preferred_element_type=_f32,
                 precision=_HI)
    aj = jnp.dot(xo, wen_ref[HID:2 * HID, :], preferred_element_type=_f32,
                 precision=_HI)
    z = jnp.zeros((xb.shape[0], 3), _f32)
    tab_ref[...] = jnp.concatenate([ai, aj, cn, z], axis=1)


def _tc_layer(F, xp, tab_prev, aggpair, Wh, bh, Wen):
    return pl.pallas_call(
        functools.partial(_layer_body, F),
        grid=(NPAD // SLAB,),
        in_specs=[
            pl.BlockSpec((SLAB, F), lambda i: (i, 0)),
            pl.BlockSpec((SLAB, TD), lambda i: (i, 0)),
            pl.BlockSpec((2, SLAB, TD), lambda i: (0, i, 0)),
            pl.BlockSpec((F + 1, HID), lambda i: (0, 0)),
            pl.BlockSpec((1, HID), lambda i: (0, 0)),
            pl.BlockSpec((2 * HID + 5, 1), lambda i: (0, 0)),
        ],
        out_specs=[
            pl.BlockSpec((SLAB, HID), lambda i: (i, 0)),
            pl.BlockSpec((SLAB, TD), lambda i: (i, 0)),
        ],
        out_shape=[
            jax.ShapeDtypeStruct((NPAD, HID), _f32),
            jax.ShapeDtypeStruct((NPAD, TD), _f32),
        ],
    )(xp, tab_prev, aggpair, Wh, bh, Wen)


def _final_body(x_ref, agg_ref, wh_ref, bh_ref, b_ref, wl_ref, bl_ref,
                wl2_ref, bl2_ref, out_ref, acc_ref):
    i = pl.program_id(0)
    xb = x_ref[...]
    aggm = agg_ref[0, :, 0:1] + agg_ref[1, :, 0:1]
    h = jnp.dot(xb, wh_ref[0:HID, :], preferred_element_type=_f32)
    h = h + aggm * wh_ref[HID:HID + 1, :] + bh_ref[...]
    x3 = jnp.maximum(h, 0.0)
    bblk = b_ref[0, 0, :]
    oh = (bblk[:, None] == lax.broadcasted_iota(_i32, (SLAB, NG), 1))
    part = lax.dot_general(oh.astype(_f32), x3, (((0,), (0,)), ((), ())),
                           preferred_element_type=_f32, precision=_HI)

    @pl.when(i == 0)
    def _():
        acc_ref[...] = part

    @pl.when(i != 0)
    def _():
        acc_ref[...] = acc_ref[...] + part

    @pl.when(i == (NPAD // SLAB) - 1)
    def _():
        pooled = acc_ref[...]
        hh = jnp.maximum(
            jnp.dot(pooled, wl_ref[...], preferred_element_type=_f32,
                    precision=_HI) + bl_ref[...], 0.0)
        out_ref[...] = (jnp.dot(hh, wl2_ref[...], preferred_element_type=_f32,
                                precision=_HI) + bl2_ref[...])


def _tc_final(x2, aggpair, Wh3, bh3, batch3d, Wl, bl, Wl2, bl2):
    return pl.pallas_call(
        _final_body,
        grid=(NPAD // SLAB,),
        in_specs=[
            pl.BlockSpec((SLAB, HID), lambda i: (i, 0)),
            pl.BlockSpec((2, SLAB, TD), lambda i: (0, i, 0)),
            pl.BlockSpec((HID + 1, HID), lambda i: (0, 0)),
            pl.BlockSpec((1, HID), lambda i: (0, 0)),
            pl.BlockSpec((1, 1, SLAB), lambda i: (i, 0, 0)),
            pl.BlockSpec((HID, NG), lambda i: (0, 0)),
            pl.BlockSpec((1, NG), lambda i: (0, 0)),
            pl.BlockSpec((NG, 1), lambda i: (0, 0)),
            pl.BlockSpec((1, 1), lambda i: (0, 0)),
        ],
        out_specs=pl.BlockSpec((NG, 1), lambda i: (0, 0)),
        out_shape=jax.ShapeDtypeStruct((NG, 1), _f32),
        scratch_shapes=[pltpu.VMEM((NG, HID), _f32)],
    )(x2, aggpair, Wh3, bh3, batch3d, Wl, bl, Wl2, bl2)


def _params16(We, Wx, bx, be, F):
    return jnp.concatenate([
        jnp.zeros((1,), _f32),
        We[2 * F + 4, :], Wx[0, :], bx,
        We[2 * F + 0, :], We[2 * F + 1, :], We[2 * F + 2, :], We[2 * F + 3, :],
        be, jnp.zeros((7,), _f32),
    ])


def kernel(x, edge_index, edge_attr, pos, batch,
           We1, be1, Wx1, bx1, Wh1, bh1,
           We2, be2, Wx2, bx2, Wh2, bh2,
           We3, be3, Wx3, bx3, Wh3, bh3,
           Wl, bl, Wl2, bl2):
    srcp = edge_index[0].astype(_i32).reshape(EROWS, 128)
    dstp = edge_index[1].astype(_i32).reshape(EROWS, 128)
    eap = edge_attr
    xp = jnp.pad(x, ((0, NPAD - N), (0, 0)))
    posp = jnp.pad(pos, ((0, NPAD - N), (0, 0)))
    batchp = jnp.concatenate(
        [batch.astype(_i32), jnp.full((NPAD - N,), NG, _i32)]
    ).reshape(NPAD // SLAB, 1, SLAB)
    zer = jnp.zeros((SLAB, TD), _f32)

    par1 = _params16(We1, Wx1, bx1, be1, 11)
    par2 = _params16(We2, Wx2, bx2, be2, HID)
    par3 = _params16(We3, Wx3, bx3, be3, HID)

    tab1 = _tc_prep(xp, posp, We1)
    agg1 = _sc_edge_pass(tab1, srcp, dstp, eap, par1, zer)
    x1, tab2 = _tc_layer(11, xp, tab1, agg1, Wh1, bh1.reshape(1, HID), We2)
    agg2 = _sc_edge_pass(tab2, srcp, dstp, eap, par2, zer)
    x2, tab3 = _tc_layer(HID, x1, tab2, agg2, Wh2, bh2.reshape(1, HID), We3)
    agg3 = _sc_edge_pass(tab3, srcp, dstp, eap, par3, zer)
    out = _tc_final(x2, agg3, Wh3, bh3.reshape(1, HID), batchp,
                    Wl, bl.reshape(1, NG), Wl2, bl2.reshape(1, 1))
    return out


# speedup vs baseline: 10.3933x; 1.0002x over previous
import functools

import jax
import jax.numpy as jnp
from jax import lax
from jax.experimental import pallas as pl
from jax.experimental.pallas import tpu as pltpu
from jax.experimental.pallas import tpu_sc as plsc

N = 50000
NG = 64
NPAD = 50048
SLAB = NPAD // 16
E = 800000
EROWS = E // 128
CR = 10
C = CR * 128
NCHUNK = EROWS // CR
KCH = 20
HID = 128
TD = 8

_f32 = jnp.float32
_i32 = jnp.int32
_HI = lax.Precision.HIGHEST


_mesh = plsc.VectorSubcoreMesh(core_axis_name="c", subcore_axis_name="s")


@functools.partial(
    pl.kernel,
    out_type=jax.ShapeDtypeStruct((2, NPAD, TD), _f32),
    mesh=_mesh,
    compiler_params=pltpu.CompilerParams(needs_layout_passes=False,
                                         use_tc_tiling_on_sc=False),
    scratch_types=[
        [pltpu.VMEM((CR, 128), _i32)] * 2,
        pltpu.VMEM((CR, 128), _i32),
        [pltpu.VMEM((C, 4), _f32)] * 2,
        [pltpu.VMEM((C, TD), _f32)] * 2,
        [pltpu.VMEM((C, TD), _f32)] * 2,
        [pltpu.VMEM((C, TD), _f32)] * 2,
        pltpu.VMEM((16,), _f32),
        pltpu.VMEM_SHARED((NPAD, TD), _f32),
        pltpu.SemaphoreType.DMA,
        pltpu.SemaphoreType.DMA,
    ],
)
def _sc_edge_pass(tab_hbm, eidx_hbm, ea_hbm, par_hbm, zer_hbm,
                  out_hbm, src_v, dst_v, ea_v, ts_v, td_v, msg_v, par_v, agg,
                  gsem, ssem):
    cid = lax.axis_index("c")
    sid = lax.axis_index("s")
    wid = cid * 16 + sid

    pltpu.sync_copy(zer_hbm.at[pl.ds(0, SLAB)], agg.at[pl.ds(sid * SLAB, SLAB)])
    pltpu.sync_copy(zer_hbm.at[pl.ds(0, C)], msg_v[0])
    pltpu.sync_copy(zer_hbm.at[pl.ds(0, C)], msg_v[1])
    pltpu.sync_copy(par_hbm, par_v)
    plsc.subcore_barrier()

    zero16 = jnp.zeros((16,), _i32)

    def bc(j):
        return plsc.load_gather(par_v, [zero16 + j])

    w_d, w_x, b_x = bc(1), bc(2), bc(3)
    wa0, wa1, wa2, wa3 = bc(4), bc(5), bc(6), bc(7)
    be = bc(8)

    iota16 = lax.iota(_i32, 16)
    c0, c1, c2, c3 = zero16, zero16 + 1, zero16 + 2, zero16 + 3
    c4 = zero16 + 4
    magic = jnp.full((16,), 0x5F3759DF, _i32)

    def chunk_of(k):
        if k == KCH - 1:
            ch = jnp.where(wid + 32 * k < NCHUNK, wid + 32 * k, wid)
            vmask = jnp.where(zero16 + (wid + 32 * k) < NCHUNK, 1.0, 0.0)
            return ch, vmask
        return wid + 32 * k, None

    def fire_chunk(k, buf):
        ch, _ = chunk_of(k)
        pltpu.sync_copy(eidx_hbm.at[0, pl.ds(ch * CR, CR)], src_v[buf])
        pltpu.sync_copy(eidx_hbm.at[1, pl.ds(ch * CR, CR)], dst_v)
        pltpu.sync_copy(ea_hbm.at[pl.ds(ch * C, C)], ea_v[buf])
        cps = []
        for j in range(CR):
            cps.append(pltpu.async_copy(
                tab_hbm.at[src_v[buf].at[j]],
                ts_v[buf].at[pl.ds(j * 128, 128)], gsem))
            cps.append(pltpu.async_copy(
                tab_hbm.at[dst_v.at[j]],
                td_v[buf].at[pl.ds(j * 128, 128)], gsem))
        return cps

    def compute_chunk(k, buf):
        _, vmask = chunk_of(k)
        msg = msg_v[buf]
        tsb = ts_v[buf]
        tdb = td_v[buf]
        eab = ea_v[buf]

        def body(i, carry):
            rows = iota16 + i * 16
            a_s = plsc.load_gather(tsb, [rows, c0])
            sx = plsc.load_gather(tsb, [rows, c2])
            sy = plsc.load_gather(tsb, [rows, c3])
            sz = plsc.load_gather(tsb, [rows, c4])
            a_d = plsc.load_gather(tdb, [rows, c1])
            tx = plsc.load_gather(tdb, [rows, c2])
            ty = plsc.load_gather(tdb, [rows, c3])
            tz = plsc.load_gather(tdb, [rows, c4])
            ea0 = plsc.load_gather(eab, [rows, c0])
            ea1 = plsc.load_gather(eab, [rows, c1])
            ea2 = plsc.load_gather(eab, [rows, c2])
            ea3 = plsc.load_gather(eab, [rows, c3])

            ec = ea0 * wa0 + ea1 * wa1 + ea2 * wa2 + ea3 * wa3
            dx = sx - tx
            dy = sy - ty
            dz = sz - tz
            d2 = dx * dx + dy * dy + dz * dz
            yi = magic - (plsc.bitcast(d2, _i32) >> 1)
            y = plsc.bitcast(yi, _f32)
            hx = 0.5 * d2
            y = y * (1.5 - hx * y * y)
            y = y * (1.5 - hx * y * y)
            y = y * (1.5 - hx * y * y)
            dist = d2 * y

            m = a_s + a_d + ec + w_d * dist + be
            info = m * w_x + b_x
            zx = dx * info
            zy = dy * info
            zz = dz * info
            if vmask is not None:
                m = m * vmask
                zx = zx * vmask
                zy = zy * vmask
                zz = zz * vmask
            plsc.store_scatter(msg, [rows, c0], m)
            plsc.store_scatter(msg, [rows, c1], zx)
            plsc.store_scatter(msg, [rows, c2], zy)
            plsc.store_scatter(msg, [rows, c3], zz)
            return carry

        lax.fori_loop(0, C // 16, body, 0)

        return [pltpu.async_copy(msg.at[pl.ds(j * 128, 128)],
                                 agg.at[src_v[buf].at[j]], ssem, add=True)
                for j in range(CR)]

    gath = {0: fire_chunk(0, 0)}
    scat = {}
    for k in range(KCH):
        buf = k % 2
        for cp in gath.pop(k):
            cp.wait()
        if k + 1 < KCH:
            if k - 1 in scat:
                for cp in scat.pop(k - 1):
                    cp.wait()
            gath[k + 1] = fire_chunk(k + 1, (k + 1) % 2)
        scat[k] = compute_chunk(k, buf)

    for _, scs in scat.items():
        for cp in scs:
            cp.wait()

    plsc.subcore_barrier()
    pltpu.sync_copy(agg.at[pl.ds(sid * SLAB, SLAB)],
                    out_hbm.at[cid, pl.ds(sid * SLAB, SLAB)])


def _prep_body(x_ref, pos_ref, we_ref, tab_ref):
    xb = x_ref[...]
    we = we_ref[...]
    ai = jnp.dot(xb, we[0:11, :], preferred_element_type=_f32, precision=_HI)
    aj = jnp.dot(xb, we[11:22, :], preferred_element_type=_f32, precision=_HI)
    p = pos_ref[...]
    z = jnp.zeros((xb.shape[0], 3), _f32)
    tab_ref[...] = jnp.concatenate([ai, aj, p, z], axis=1)


def _tc_prep(xp, posp, We1):
    return pl.pallas_call(
        _prep_body,
        grid=(NPAD // SLAB,),
        in_specs=[
            pl.BlockSpec((SLAB, 11), lambda i: (i, 0)),
            pl.BlockSpec((SLAB, 3), lambda i: (i, 0)),
            pl.BlockSpec((27, 1), lambda i: (0, 0)),
        ],
        out_specs=pl.BlockSpec((SLAB, TD), lambda i: (i, 0)),
        out_shape=jax.ShapeDtypeStruct((NPAD, TD), _f32),
    )(xp, posp, We1)


def _layer_body(F, x_ref, tabp_ref, agg_ref, wh_ref, bh_ref, wen_ref,
                xo_ref, tab_ref):
    xb = x_ref[...]
    aggm = agg_ref[0, :, 0:1] + agg_ref[1, :, 0:1]
    aggz = agg_ref[0, :, 1:4] + agg_ref[1, :, 1:4]
    h = jnp.dot(xb, wh_ref[0:F, :], preferred_element_type=_f32)
    h = h + aggm * wh_ref[F:F + 1, :] + bh_ref[...]
    xo = jnp.maximum(h, 0.0)
    xo_ref[...] = xo
    cn = tabp_ref[:, 2:5] + aggz * (1.0 / N)
    ai = jnp.dot(xo, wen_ref[0:HID, :], preferred_element_type=_f32,
                 precision=_HI)
    aj = jnp.dot(xo, wen_ref[HID:2 * HID, :], preferred_element_type=_f32,
                 precision=_HI)
    z = jnp.zeros((xb.shape[0], 3), _f32)
    tab_ref[...] = jnp.concatenate([ai, aj, cn, z], axis=1)


def _tc_layer(F, xp, tab_prev, aggpair, Wh, bh, Wen):
    return pl.pallas_call(
        functools.partial(_layer_body, F),
        grid=(NPAD // SLAB,),
        in_specs=[
            pl.BlockSpec((SLAB, F), lambda i: (i, 0)),
            pl.BlockSpec((SLAB, TD), lambda i: (i, 0)),
            pl.BlockSpec((2, SLAB, TD), lambda i: (0, i, 0)),
            pl.BlockSpec((F + 1, HID), lambda i: (0, 0)),
            pl.BlockSpec((1, HID), lambda i: (0, 0)),
            pl.BlockSpec((2 * HID + 5, 1), lambda i: (0, 0)),
        ],
        out_specs=[
            pl.BlockSpec((SLAB, HID), lambda i: (i, 0)),
            pl.BlockSpec((SLAB, TD), lambda i: (i, 0)),
        ],
        out_shape=[
            jax.ShapeDtypeStruct((NPAD, HID), _f32),
            jax.ShapeDtypeStruct((NPAD, TD), _f32),
        ],
    )(xp, tab_prev, aggpair, Wh, bh, Wen)


def _final_body(x_ref, agg_ref, wh_ref, bh_ref, b_ref, wl_ref, bl_ref,
                wl2_ref, bl2_ref, out_ref, acc_ref):
    i = pl.program_id(0)
    xb = x_ref[...]
    aggm = agg_ref[0, :, 0:1] + agg_ref[1, :, 0:1]
    h = jnp.dot(xb, wh_ref[0:HID, :], preferred_element_type=_f32)
    h = h + aggm * wh_ref[HID:HID + 1, :] + bh_ref[...]
    x3 = jnp.maximum(h, 0.0)
    bblk = b_ref[0, 0, :]
    oh = (bblk[:, None] == lax.broadcasted_iota(_i32, (SLAB, NG), 1))
    part = lax.dot_general(oh.astype(_f32), x3, (((0,), (0,)), ((), ())),
                           preferred_element_type=_f32, precision=_HI)

    @pl.when(i == 0)
    def _():
        acc_ref[...] = part

    @pl.when(i != 0)
    def _():
        acc_ref[...] = acc_ref[...] + part

    @pl.when(i == (NPAD // SLAB) - 1)
    def _():
        pooled = acc_ref[...]
        hh = jnp.maximum(
            jnp.dot(pooled, wl_ref[...], preferred_element_type=_f32,
                    precision=_HI) + bl_ref[...], 0.0)
        out_ref[...] = (jnp.dot(hh, wl2_ref[...], preferred_element_type=_f32,
                                precision=_HI) + bl2_ref[...])


def _tc_final(x2, aggpair, Wh3, bh3, batch3d, Wl, bl, Wl2, bl2):
    return pl.pallas_call(
        _final_body,
        grid=(NPAD // SLAB,),
        in_specs=[
            pl.BlockSpec((SLAB, HID), lambda i: (i, 0)),
            pl.BlockSpec((2, SLAB, TD), lambda i: (0, i, 0)),
            pl.BlockSpec((HID + 1, HID), lambda i: (0, 0)),
            pl.BlockSpec((1, HID), lambda i: (0, 0)),
            pl.BlockSpec((1, 1, SLAB), lambda i: (i, 0, 0)),
            pl.BlockSpec((HID, NG), lambda i: (0, 0)),
            pl.BlockSpec((1, NG), lambda i: (0, 0)),
            pl.BlockSpec((NG, 1), lambda i: (0, 0)),
            pl.BlockSpec((1, 1), lambda i: (0, 0)),
        ],
        out_specs=pl.BlockSpec((NG, 1), lambda i: (0, 0)),
        out_shape=jax.ShapeDtypeStruct((NG, 1), _f32),
        scratch_shapes=[pltpu.VMEM((NG, HID), _f32)],
    )(x2, aggpair, Wh3, bh3, batch3d, Wl, bl, Wl2, bl2)


def _params16(We, Wx, bx, be, F):
    return jnp.concatenate([
        jnp.zeros((1,), _f32),
        We[2 * F + 4, :], Wx[0, :], bx,
        We[2 * F + 0, :], We[2 * F + 1, :], We[2 * F + 2, :], We[2 * F + 3, :],
        be, jnp.zeros((7,), _f32),
    ])


def kernel(x, edge_index, edge_attr, pos, batch,
           We1, be1, Wx1, bx1, Wh1, bh1,
           We2, be2, Wx2, bx2, Wh2, bh2,
           We3, be3, Wx3, bx3, Wh3, bh3,
           Wl, bl, Wl2, bl2):
    eidx = edge_index.astype(_i32).reshape(2, EROWS, 128)
    eap = edge_attr
    xp = jnp.pad(x, ((0, NPAD - N), (0, 0)))
    posp = jnp.pad(pos, ((0, NPAD - N), (0, 0)))
    batchp = jnp.concatenate(
        [batch.astype(_i32), jnp.full((NPAD - N,), NG, _i32)]
    ).reshape(NPAD // SLAB, 1, SLAB)
    zer = jnp.zeros((SLAB, TD), _f32)

    par1 = _params16(We1, Wx1, bx1, be1, 11)
    par2 = _params16(We2, Wx2, bx2, be2, HID)
    par3 = _params16(We3, Wx3, bx3, be3, HID)

    tab1 = _tc_prep(xp, posp, We1)
    agg1 = _sc_edge_pass(tab1, eidx, eap, par1, zer)
    x1, tab2 = _tc_layer(11, xp, tab1, agg1, Wh1, bh1.reshape(1, HID), We2)
    agg2 = _sc_edge_pass(tab2, eidx, eap, par2, zer)
    x2, tab3 = _tc_layer(HID, x1, tab2, agg2, Wh2, bh2.reshape(1, HID), We3)
    agg3 = _sc_edge_pass(tab3, eidx, eap, par3, zer)
    out = _tc_final(x2, agg3, Wh3, bh3.reshape(1, HID), batchp,
                    Wl, bl.reshape(1, NG), Wl2, bl2.reshape(1, 1))
    return out


# speedup vs baseline: 20.4024x; 1.9630x over previous
import functools

import jax
import jax.numpy as jnp
from jax import lax
from jax.experimental import pallas as pl
from jax.experimental.pallas import tpu as pltpu
from jax.experimental.pallas import tpu_sc as plsc

N = 50000
NG = 64
NPAD = 50048
SLAB = NPAD // 16
E = 800000
EROWS = E // 128
CR = 10
C = CR * 128
NCHUNK = EROWS // CR
KCH = 20
HID = 128
TD = 8

_f32 = jnp.float32
_i32 = jnp.int32
_HI = lax.Precision.HIGHEST


_mesh = plsc.VectorSubcoreMesh(core_axis_name="c", subcore_axis_name="s")


@functools.partial(
    pl.kernel,
    out_type=jax.ShapeDtypeStruct((2, NPAD, TD), _f32),
    mesh=_mesh,
    compiler_params=pltpu.CompilerParams(needs_layout_passes=False,
                                         use_tc_tiling_on_sc=False),
    scratch_types=[
        [pltpu.VMEM((CR, 128), _i32)] * 2,
        pltpu.VMEM((CR, 128), _i32),
        [pltpu.VMEM((4, C), _f32)] * 2,
        [pltpu.VMEM((C, TD), _f32)] * 2,
        [pltpu.VMEM((C, TD), _f32)] * 2,
        [pltpu.VMEM((C, TD), _f32)] * 2,
        pltpu.VMEM((16,), _f32),
        pltpu.VMEM_SHARED((NPAD, TD), _f32),
        pltpu.SemaphoreType.DMA,
        pltpu.SemaphoreType.DMA,
    ],
)
def _sc_edge_pass(tab_hbm, eidx_hbm, ea_hbm, par_hbm, zer_hbm,
                  out_hbm, src_v, dst_v, ea_v, ts_v, td_v, msg_v, par_v, agg,
                  gsem, ssem):
    cid = lax.axis_index("c")
    sid = lax.axis_index("s")
    wid = cid * 16 + sid

    pltpu.sync_copy(zer_hbm.at[pl.ds(0, SLAB)], agg.at[pl.ds(sid * SLAB, SLAB)])
    pltpu.sync_copy(zer_hbm.at[pl.ds(0, C)], msg_v[0])
    pltpu.sync_copy(zer_hbm.at[pl.ds(0, C)], msg_v[1])
    pltpu.sync_copy(par_hbm, par_v)
    plsc.subcore_barrier()

    zero16 = jnp.zeros((16,), _i32)

    def bc(j):
        return plsc.load_gather(par_v, [zero16 + j])

    w_d, w_x, b_x = bc(1), bc(2), bc(3)
    wa0, wa1, wa2, wa3 = bc(4), bc(5), bc(6), bc(7)
    be = bc(8)

    iota16 = lax.iota(_i32, 16)
    c0, c1, c2, c3 = zero16, zero16 + 1, zero16 + 2, zero16 + 3
    c4 = zero16 + 4
    magic = jnp.full((16,), 0x5F3759DF, _i32)

    def chunk_of(k):
        if k == KCH - 1:
            ch = jnp.where(wid + 32 * k < NCHUNK, wid + 32 * k, wid)
            vmask = jnp.where(zero16 + (wid + 32 * k) < NCHUNK, 1.0, 0.0)
            return ch, vmask
        return wid + 32 * k, None

    def fire_chunk(k, buf):
        ch, _ = chunk_of(k)
        pltpu.sync_copy(eidx_hbm.at[0, pl.ds(ch * CR, CR)], src_v[buf])
        pltpu.sync_copy(eidx_hbm.at[1, pl.ds(ch * CR, CR)], dst_v)
        for q in range(4):
            pltpu.sync_copy(ea_hbm.at[q, pl.ds(ch * C, C)],
                            ea_v[buf].at[q])
        cps = []
        for j in range(CR):
            cps.append(pltpu.async_copy(
                tab_hbm.at[src_v[buf].at[j]],
                ts_v[buf].at[pl.ds(j * 128, 128)], gsem))
            cps.append(pltpu.async_copy(
                tab_hbm.at[dst_v.at[j]],
                td_v[buf].at[pl.ds(j * 128, 128)], gsem))
        return cps

    def compute_chunk(k, buf):
        _, vmask = chunk_of(k)
        msg = msg_v[buf]
        tsb = ts_v[buf]
        tdb = td_v[buf]
        eab = ea_v[buf]

        def body(i, carry):
            rows = iota16 + i * 16
            a_s = plsc.load_gather(tsb, [rows, c0])
            sx = plsc.load_gather(tsb, [rows, c2])
            sy = plsc.load_gather(tsb, [rows, c3])
            sz = plsc.load_gather(tsb, [rows, c4])
            a_d = plsc.load_gather(tdb, [rows, c1])
            tx = plsc.load_gather(tdb, [rows, c2])
            ty = plsc.load_gather(tdb, [rows, c3])
            tz = plsc.load_gather(tdb, [rows, c4])
            ea0 = eab[0, pl.ds(i * 16, 16)]
            ea1 = eab[1, pl.ds(i * 16, 16)]
            ea2 = eab[2, pl.ds(i * 16, 16)]
            ea3 = eab[3, pl.ds(i * 16, 16)]

            ec = ea0 * wa0 + ea1 * wa1 + ea2 * wa2 + ea3 * wa3
            dx = sx - tx
            dy = sy - ty
            dz = sz - tz
            d2 = dx * dx + dy * dy + dz * dz
            yi = magic - (plsc.bitcast(d2, _i32) >> 1)
            y = plsc.bitcast(yi, _f32)
            hx = 0.5 * d2
            y = y * (1.5 - hx * y * y)
            y = y * (1.5 - hx * y * y)
            y = y * (1.5 - hx * y * y)
            dist = d2 * y

            m = a_s + a_d + ec + w_d * dist + be
            info = m * w_x + b_x
            zx = dx * info
            zy = dy * info
            zz = dz * info
            if vmask is not None:
                m = m * vmask
                zx = zx * vmask
                zy = zy * vmask
                zz = zz * vmask
            plsc.store_scatter(msg, [rows, c0], m)
            plsc.store_scatter(msg, [rows, c1], zx)
            plsc.store_scatter(msg, [rows, c2], zy)
            plsc.store_scatter(msg, [rows, c3], zz)
            return carry

        lax.fori_loop(0, C // 16, body, 0)

        return [pltpu.async_copy(msg.at[pl.ds(j * 128, 128)],
                                 agg.at[src_v[buf].at[j]], ssem, add=True)
                for j in range(CR)]

    gath = {0: fire_chunk(0, 0)}
    scat = {}
    for k in range(KCH):
        buf = k % 2
        for cp in gath.pop(k):
            cp.wait()
        if k + 1 < KCH:
            if k - 1 in scat:
                for cp in scat.pop(k - 1):
                    cp.wait()
            gath[k + 1] = fire_chunk(k + 1, (k + 1) % 2)
        scat[k] = compute_chunk(k, buf)

    for _, scs in scat.items():
        for cp in scs:
            cp.wait()

    plsc.subcore_barrier()
    pltpu.sync_copy(agg.at[pl.ds(sid * SLAB, SLAB)],
                    out_hbm.at[cid, pl.ds(sid * SLAB, SLAB)])


def _prep_body(x_ref, pos_ref, we_ref, tab_ref):
    xb = x_ref[...]
    we = we_ref[...]
    ai = jnp.dot(xb, we[0:11, :], preferred_element_type=_f32, precision=_HI)
    aj = jnp.dot(xb, we[11:22, :], preferred_element_type=_f32, precision=_HI)
    p = pos_ref[...]
    z = jnp.zeros((xb.shape[0], 3), _f32)
    tab_ref[...] = jnp.concatenate([ai, aj, p, z], axis=1)


def _tc_prep(xp, posp, We1):
    return pl.pallas_call(
        _prep_body,
        grid=(NPAD // SLAB,),
        in_specs=[
            pl.BlockSpec((SLAB, 11), lambda i: (i, 0)),
            pl.BlockSpec((SLAB, 3), lambda i: (i, 0)),
            pl.BlockSpec((27, 1), lambda i: (0, 0)),
        ],
        out_specs=pl.BlockSpec((SLAB, TD), lambda i: (i, 0)),
        out_shape=jax.ShapeDtypeStruct((NPAD, TD), _f32),
    )(xp, posp, We1)


def _layer_body(F, x_ref, tabp_ref, agg_ref, wh_ref, bh_ref, wen_ref,
                xo_ref, tab_ref):
    xb = x_ref[...]
    aggm = agg_ref[0, :, 0:1] + agg_ref[1, :, 0:1]
    aggz = agg_ref[0, :, 1:4] + agg_ref[1, :, 1:4]
    h = jnp.dot(xb, wh_ref[0:F, :], preferred_element_type=_f32)
    h = h + aggm * wh_ref[F:F + 1, :] + bh_ref[...]
    xo = jnp.maximum(h, 0.0)
    xo_ref[...] = xo
    cn = tabp_ref[:, 2:5] + aggz * (1.0 / N)
    ai = jnp.dot(xo, wen_ref[0:HID, :], preferred_element_type=_f32,
                 precision=_HI)
    aj = jnp.dot(xo, wen_ref[HID:2 * HID, :], preferred_element_type=_f32,
                 precision=_HI)
    z = jnp.zeros((xb.shape[0], 3), _f32)
    tab_ref[...] = jnp.concatenate([ai, aj, cn, z], axis=1)


def _tc_layer(F, xp, tab_prev, aggpair, Wh, bh, Wen):
    return pl.pallas_call(
        functools.partial(_layer_body, F),
        grid=(NPAD // SLAB,),
        in_specs=[
            pl.BlockSpec((SLAB, F), lambda i: (i, 0)),
            pl.BlockSpec((SLAB, TD), lambda i: (i, 0)),
            pl.BlockSpec((2, SLAB, TD), lambda i: (0, i, 0)),
            pl.BlockSpec((F + 1, HID), lambda i: (0, 0)),
            pl.BlockSpec((1, HID), lambda i: (0, 0)),
            pl.BlockSpec((2 * HID + 5, 1), lambda i: (0, 0)),
        ],
        out_specs=[
            pl.BlockSpec((SLAB, HID), lambda i: (i, 0)),
            pl.BlockSpec((SLAB, TD), lambda i: (i, 0)),
        ],
        out_shape=[
            jax.ShapeDtypeStruct((NPAD, HID), _f32),
            jax.ShapeDtypeStruct((NPAD, TD), _f32),
        ],
    )(xp, tab_prev, aggpair, Wh, bh, Wen)


def _final_body(x_ref, agg_ref, wh_ref, bh_ref, b_ref, wl_ref, bl_ref,
                wl2_ref, bl2_ref, out_ref, acc_ref):
    i = pl.program_id(0)
    xb = x_ref[...]
    aggm = agg_ref[0, :, 0:1] + agg_ref[1, :, 0:1]
    h = jnp.dot(xb, wh_ref[0:HID, :], preferred_element_type=_f32)
    h = h + aggm * wh_ref[HID:HID + 1, :] + bh_ref[...]
    x3 = jnp.maximum(h, 0.0)
    bblk = b_ref[0, 0, :]
    oh = (bblk[:, None] == lax.broadcasted_iota(_i32, (SLAB, NG), 1))
    part = lax.dot_general(oh.astype(_f32), x3, (((0,), (0,)), ((), ())),
                           preferred_element_type=_f32, precision=_HI)

    @pl.when(i == 0)
    def _():
        acc_ref[...] = part

    @pl.when(i != 0)
    def _():
        acc_ref[...] = acc_ref[...] + part

    @pl.when(i == (NPAD // SLAB) - 1)
    def _():
        pooled = acc_ref[...]
        hh = jnp.maximum(
            jnp.dot(pooled, wl_ref[...], preferred_element_type=_f32,
                    precision=_HI) + bl_ref[...], 0.0)
        out_ref[...] = (jnp.dot(hh, wl2_ref[...], preferred_element_type=_f32,
                                precision=_HI) + bl2_ref[...])


def _tc_final(x2, aggpair, Wh3, bh3, batch3d, Wl, bl, Wl2, bl2):
    return pl.pallas_call(
        _final_body,
        grid=(NPAD // SLAB,),
        in_specs=[
            pl.BlockSpec((SLAB, HID), lambda i: (i, 0)),
            pl.BlockSpec((2, SLAB, TD), lambda i: (0, i, 0)),
            pl.BlockSpec((HID + 1, HID), lambda i: (0, 0)),
            pl.BlockSpec((1, HID), lambda i: (0, 0)),
            pl.BlockSpec((1, 1, SLAB), lambda i: (i, 0, 0)),
            pl.BlockSpec((HID, NG), lambda i: (0, 0)),
            pl.BlockSpec((1, NG), lambda i: (0, 0)),
            pl.BlockSpec((NG, 1), lambda i: (0, 0)),
            pl.BlockSpec((1, 1), lambda i: (0, 0)),
        ],
        out_specs=pl.BlockSpec((NG, 1), lambda i: (0, 0)),
        out_shape=jax.ShapeDtypeStruct((NG, 1), _f32),
        scratch_shapes=[pltpu.VMEM((NG, HID), _f32)],
    )(x2, aggpair, Wh3, bh3, batch3d, Wl, bl, Wl2, bl2)


def _params16(We, Wx, bx, be, F):
    return jnp.concatenate([
        jnp.zeros((1,), _f32),
        We[2 * F + 4, :], Wx[0, :], bx,
        We[2 * F + 0, :], We[2 * F + 1, :], We[2 * F + 2, :], We[2 * F + 3, :],
        be, jnp.zeros((7,), _f32),
    ])


def kernel(x, edge_index, edge_attr, pos, batch,
           We1, be1, Wx1, bx1, Wh1, bh1,
           We2, be2, Wx2, bx2, Wh2, bh2,
           We3, be3, Wx3, bx3, Wh3, bh3,
           Wl, bl, Wl2, bl2):
    eidx = edge_index.astype(_i32).reshape(2, EROWS, 128)
    eap = edge_attr.T
    xp = jnp.pad(x, ((0, NPAD - N), (0, 0)))
    posp = jnp.pad(pos, ((0, NPAD - N), (0, 0)))
    batchp = jnp.concatenate(
        [batch.astype(_i32), jnp.full((NPAD - N,), NG, _i32)]
    ).reshape(NPAD // SLAB, 1, SLAB)
    zer = jnp.zeros((SLAB, TD), _f32)

    par1 = _params16(We1, Wx1, bx1, be1, 11)
    par2 = _params16(We2, Wx2, bx2, be2, HID)
    par3 = _params16(We3, Wx3, bx3, be3, HID)

    tab1 = _tc_prep(xp, posp, We1)
    agg1 = _sc_edge_pass(tab1, eidx, eap, par1, zer)
    x1, tab2 = _tc_layer(11, xp, tab1, agg1, Wh1, bh1.reshape(1, HID), We2)
    agg2 = _sc_edge_pass(tab2, eidx, eap, par2, zer)
    x2, tab3 = _tc_layer(HID, x1, tab2, agg2, Wh2, bh2.reshape(1, HID), We3)
    agg3 = _sc_edge_pass(tab3, eidx, eap, par3, zer)
    out = _tc_final(x2, agg3, Wh3, bh3.reshape(1, HID), batchp,
                    Wl, bl.reshape(1, NG), Wl2, bl2.reshape(1, 1))
    return out


# speedup vs baseline: 21.8205x; 1.0695x over previous
import functools

import jax
import jax.numpy as jnp
from jax import lax
from jax.experimental import pallas as pl
from jax.experimental.pallas import tpu as pltpu
from jax.experimental.pallas import tpu_sc as plsc

N = 50000
NG = 64
NPAD = 50048
SLAB = NPAD // 16
E = 800000
EROWS = E // 128
CR = 10
C = CR * 128
NCHUNK = EROWS // CR
KCH = 20
HID = 128
TD = 8

_f32 = jnp.float32
_i32 = jnp.int32
_HI = lax.Precision.HIGHEST


_mesh = plsc.VectorSubcoreMesh(core_axis_name="c", subcore_axis_name="s")


@functools.partial(
    pl.kernel,
    out_type=jax.ShapeDtypeStruct((2, NPAD, TD), _f32),
    mesh=_mesh,
    compiler_params=pltpu.CompilerParams(needs_layout_passes=False,
                                         use_tc_tiling_on_sc=False),
    scratch_types=[
        [pltpu.VMEM((CR, 128), _i32)] * 2,
        pltpu.VMEM((CR, 128), _i32),
        [pltpu.VMEM((4, C), _f32)] * 2,
        [pltpu.VMEM((C, TD), _f32)] * 2,
        [pltpu.VMEM((C, TD), _f32)] * 2,
        [pltpu.VMEM((C, TD), _f32)] * 2,
        pltpu.VMEM((16,), _f32),
        pltpu.VMEM_SHARED((NPAD, TD), _f32),
        pltpu.SemaphoreType.DMA,
        pltpu.SemaphoreType.DMA,
    ],
)
def _sc_edge_pass(tab_hbm, eidx_hbm, ea_hbm, par_hbm, zer_hbm,
                  out_hbm, src_v, dst_v, ea_v, ts_v, td_v, msg_v, par_v, agg,
                  gsem, ssem):
    cid = lax.axis_index("c")
    sid = lax.axis_index("s")
    wid = cid * 16 + sid

    pltpu.sync_copy(zer_hbm.at[pl.ds(0, SLAB)], agg.at[pl.ds(sid * SLAB, SLAB)])
    pltpu.sync_copy(zer_hbm.at[pl.ds(0, C)], msg_v[0])
    pltpu.sync_copy(zer_hbm.at[pl.ds(0, C)], msg_v[1])
    pltpu.sync_copy(par_hbm, par_v)
    plsc.subcore_barrier()

    zero16 = jnp.zeros((16,), _i32)

    def bc(j):
        return plsc.load_gather(par_v, [zero16 + j])

    w_d, w_x, b_x = bc(1), bc(2), bc(3)
    wa0, wa1, wa2, wa3 = bc(4), bc(5), bc(6), bc(7)
    be = bc(8)

    iota16 = lax.iota(_i32, 16)
    c0, c1, c2, c3 = zero16, zero16 + 1, zero16 + 2, zero16 + 3
    c4 = zero16 + 4
    magic = jnp.full((16,), 0x5F3759DF, _i32)

    def chunk_of(k):
        if k == KCH - 1:
            ch = jnp.where(wid + 32 * k < NCHUNK, wid + 32 * k, wid)
            vmask = jnp.where(zero16 + (wid + 32 * k) < NCHUNK, 1.0, 0.0)
            return ch, vmask
        return wid + 32 * k, None

    def fire_chunk(k, buf):
        ch, _ = chunk_of(k)
        pltpu.sync_copy(eidx_hbm.at[0, pl.ds(ch * CR, CR)], src_v[buf])
        pltpu.sync_copy(eidx_hbm.at[1, pl.ds(ch * CR, CR)], dst_v)
        for q in range(4):
            pltpu.sync_copy(ea_hbm.at[q, pl.ds(ch * C, C)],
                            ea_v[buf].at[q])
        cps = []
        for j in range(CR):
            cps.append(pltpu.async_copy(
                tab_hbm.at[src_v[buf].at[j]],
                ts_v[buf].at[pl.ds(j * 128, 128)], gsem))
            cps.append(pltpu.async_copy(
                tab_hbm.at[dst_v.at[j]],
                td_v[buf].at[pl.ds(j * 128, 128)], gsem))
        return cps

    def compute_chunk(k, buf):
        _, vmask = chunk_of(k)
        msg = msg_v[buf]
        tsb = ts_v[buf]
        tdb = td_v[buf]
        eab = ea_v[buf]

        @plsc.parallel_loop(0, C // 16, 1, unroll=2)
        def body(i):
            rows = iota16 + i * 16
            a_s = plsc.load_gather(tsb, [rows, c0])
            sx = plsc.load_gather(tsb, [rows, c2])
            sy = plsc.load_gather(tsb, [rows, c3])
            sz = plsc.load_gather(tsb, [rows, c4])
            a_d = plsc.load_gather(tdb, [rows, c1])
            tx = plsc.load_gather(tdb, [rows, c2])
            ty = plsc.load_gather(tdb, [rows, c3])
            tz = plsc.load_gather(tdb, [rows, c4])
            ea0 = eab[0, pl.ds(i * 16, 16)]
            ea1 = eab[1, pl.ds(i * 16, 16)]
            ea2 = eab[2, pl.ds(i * 16, 16)]
            ea3 = eab[3, pl.ds(i * 16, 16)]

            ec = ea0 * wa0 + ea1 * wa1 + ea2 * wa2 + ea3 * wa3
            dx = sx - tx
            dy = sy - ty
            dz = sz - tz
            d2 = dx * dx + dy * dy + dz * dz
            yi = magic - (plsc.bitcast(d2, _i32) >> 1)
            y = plsc.bitcast(yi, _f32)
            hx = 0.5 * d2
            y = y * (1.5 - hx * y * y)
            y = y * (1.5 - hx * y * y)
            y = y * (1.5 - hx * y * y)
            dist = d2 * y

            m = a_s + a_d + ec + w_d * dist + be
            info = m * w_x + b_x
            zx = dx * info
            zy = dy * info
            zz = dz * info
            if vmask is not None:
                m = m * vmask
                zx = zx * vmask
                zy = zy * vmask
                zz = zz * vmask
            plsc.store_scatter(msg, [rows, c0], m)
            plsc.store_scatter(msg, [rows, c1], zx)
            plsc.store_scatter(msg, [rows, c2], zy)
            plsc.store_scatter(msg, [rows, c3], zz)


        return [pltpu.async_copy(msg.at[pl.ds(j * 128, 128)],
                                 agg.at[src_v[buf].at[j]], ssem, add=True)
                for j in range(CR)]

    gath = {0: fire_chunk(0, 0)}
    scat = {}
    for k in range(KCH):
        buf = k % 2
        for cp in gath.pop(k):
            cp.wait()
        if k + 1 < KCH:
            if k - 1 in scat:
                for cp in scat.pop(k - 1):
                    cp.wait()
            gath[k + 1] = fire_chunk(k + 1, (k + 1) % 2)
        scat[k] = compute_chunk(k, buf)

    for _, scs in scat.items():
        for cp in scs:
            cp.wait()

    plsc.subcore_barrier()
    pltpu.sync_copy(agg.at[pl.ds(sid * SLAB, SLAB)],
                    out_hbm.at[cid, pl.ds(sid * SLAB, SLAB)])


def _prep_body(x_ref, pos_ref, w1_ref, w2_ref, tab_ref):
    tab_ref[...] = (
        jnp.dot(x_ref[...], w1_ref[...], preferred_element_type=_f32,
                precision=_HI)
        + jnp.dot(pos_ref[...], w2_ref[...], preferred_element_type=_f32,
                  precision=_HI))


def _tc_prep(xp, posp, w1, w2):
    return pl.pallas_call(
        _prep_body,
        grid=(NPAD // SLAB,),
        in_specs=[
            pl.BlockSpec((SLAB, 11), lambda i: (i, 0)),
            pl.BlockSpec((SLAB, 3), lambda i: (i, 0)),
            pl.BlockSpec((11, TD), lambda i: (0, 0)),
            pl.BlockSpec((3, TD), lambda i: (0, 0)),
        ],
        out_specs=pl.BlockSpec((SLAB, TD), lambda i: (i, 0)),
        out_shape=jax.ShapeDtypeStruct((NPAD, TD), _f32),
    )(xp, posp, w1, w2)


def _layer_body(F, x_ref, tabp_ref, agg_ref, wh_ref, bh_ref, wij_ref,
                xo_ref, tab_ref):
    xb = x_ref[...]
    aggm = agg_ref[0, :, 0:1] + agg_ref[1, :, 0:1]
    aggz = agg_ref[0, :, 1:4] + agg_ref[1, :, 1:4]
    h = jnp.dot(xb, wh_ref[0:F, :], preferred_element_type=_f32)
    h = h + aggm * wh_ref[F:F + 1, :] + bh_ref[...]
    xo = jnp.maximum(h, 0.0)
    xo_ref[...] = xo
    cn = tabp_ref[:, 2:5] + aggz * (1.0 / N)
    aiaj = jnp.dot(xo, wij_ref[...], preferred_element_type=_f32,
                   precision=_HI)
    z = jnp.zeros((xb.shape[0], 3), _f32)
    tab_ref[...] = jnp.concatenate([aiaj, cn, z], axis=1)


def _tc_layer(F, xp, tab_prev, aggpair, Wh, bh, wij):
    return pl.pallas_call(
        functools.partial(_layer_body, F),
        grid=(NPAD // SLAB,),
        in_specs=[
            pl.BlockSpec((SLAB, F), lambda i: (i, 0)),
            pl.BlockSpec((SLAB, TD), lambda i: (i, 0)),
            pl.BlockSpec((2, SLAB, TD), lambda i: (0, i, 0)),
            pl.BlockSpec((F + 1, HID), lambda i: (0, 0)),
            pl.BlockSpec((1, HID), lambda i: (0, 0)),
            pl.BlockSpec((HID, 2), lambda i: (0, 0)),
        ],
        out_specs=[
            pl.BlockSpec((SLAB, HID), lambda i: (i, 0)),
            pl.BlockSpec((SLAB, TD), lambda i: (i, 0)),
        ],
        out_shape=[
            jax.ShapeDtypeStruct((NPAD, HID), _f32),
            jax.ShapeDtypeStruct((NPAD, TD), _f32),
        ],
    )(xp, tab_prev, aggpair, Wh, bh, wij)


def _final_body(x_ref, agg_ref, wh_ref, bh_ref, b_ref, wl_ref, bl_ref,
                wl2_ref, bl2_ref, out_ref, acc_ref):
    i = pl.program_id(0)
    xb = x_ref[...]
    aggm = agg_ref[0, :, 0:1] + agg_ref[1, :, 0:1]
    h = jnp.dot(xb, wh_ref[0:HID, :], preferred_element_type=_f32)
    h = h + aggm * wh_ref[HID:HID + 1, :] + bh_ref[...]
    x3 = jnp.maximum(h, 0.0)
    bblk = b_ref[0, 0, :]
    oh = (bblk[:, None] == lax.broadcasted_iota(_i32, (SLAB, NG), 1))
    part = lax.dot_general(oh.astype(_f32), x3, (((0,), (0,)), ((), ())),
                           preferred_element_type=_f32, precision=_HI)

    @pl.when(i == 0)
    def _():
        acc_ref[...] = part

    @pl.when(i != 0)
    def _():
        acc_ref[...] = acc_ref[...] + part

    @pl.when(i == (NPAD // SLAB) - 1)
    def _():
        pooled = acc_ref[...]
        hh = jnp.maximum(
            jnp.dot(pooled, wl_ref[...], preferred_element_type=_f32,
                    precision=_HI) + bl_ref[...], 0.0)
        out_ref[...] = (jnp.dot(hh, wl2_ref[...], preferred_element_type=_f32,
                                precision=_HI) + bl2_ref[...])


def _tc_final(x2, aggpair, Wh3, bh3, batch3d, Wl, bl, Wl2, bl2):
    return pl.pallas_call(
        _final_body,
        grid=(NPAD // SLAB,),
        in_specs=[
            pl.BlockSpec((SLAB, HID), lambda i: (i, 0)),
            pl.BlockSpec((2, SLAB, TD), lambda i: (0, i, 0)),
            pl.BlockSpec((HID + 1, HID), lambda i: (0, 0)),
            pl.BlockSpec((1, HID), lambda i: (0, 0)),
            pl.BlockSpec((1, 1, SLAB), lambda i: (i, 0, 0)),
            pl.BlockSpec((HID, NG), lambda i: (0, 0)),
            pl.BlockSpec((1, NG), lambda i: (0, 0)),
            pl.BlockSpec((NG, 1), lambda i: (0, 0)),
            pl.BlockSpec((1, 1), lambda i: (0, 0)),
        ],
        out_specs=pl.BlockSpec((NG, 1), lambda i: (0, 0)),
        out_shape=jax.ShapeDtypeStruct((NG, 1), _f32),
        scratch_shapes=[pltpu.VMEM((NG, HID), _f32)],
    )(x2, aggpair, Wh3, bh3, batch3d, Wl, bl, Wl2, bl2)


def _params16(We, Wx, bx, be, F):
    return jnp.concatenate([
        jnp.zeros((1,), _f32),
        We[2 * F + 4, :], Wx[0, :], bx,
        We[2 * F + 0, :], We[2 * F + 1, :], We[2 * F + 2, :], We[2 * F + 3, :],
        be, jnp.zeros((7,), _f32),
    ])


def kernel(x, edge_index, edge_attr, pos, batch,
           We1, be1, Wx1, bx1, Wh1, bh1,
           We2, be2, Wx2, bx2, Wh2, bh2,
           We3, be3, Wx3, bx3, Wh3, bh3,
           Wl, bl, Wl2, bl2):
    eidx = edge_index.astype(_i32).reshape(2, EROWS, 128)
    eap = edge_attr.T
    xp = jnp.pad(x, ((0, NPAD - N), (0, 0)))
    posp = jnp.pad(pos, ((0, NPAD - N), (0, 0)))
    batchp = jnp.concatenate(
        [batch.astype(_i32), jnp.full((NPAD - N,), NG, _i32)]
    ).reshape(NPAD // SLAB, 1, SLAB)
    zer = jnp.zeros((SLAB, TD), _f32)

    par1 = _params16(We1, Wx1, bx1, be1, 11)
    par2 = _params16(We2, Wx2, bx2, be2, HID)
    par3 = _params16(We3, Wx3, bx3, be3, HID)

    zc3 = jnp.zeros((11, 3), _f32)
    w1 = jnp.concatenate([We1[0:11], We1[11:22], zc3, zc3], axis=1)
    w2 = jnp.concatenate([jnp.zeros((3, 2), _f32), jnp.eye(3, dtype=_f32),
                          jnp.zeros((3, 3), _f32)], axis=1)
    wij2 = jnp.concatenate([We2[0:HID], We2[HID:2 * HID]], axis=1)
    wij3 = jnp.concatenate([We3[0:HID], We3[HID:2 * HID]], axis=1)

    tab1 = _tc_prep(xp, posp, w1, w2)
    agg1 = _sc_edge_pass(tab1, eidx, eap, par1, zer)
    x1, tab2 = _tc_layer(11, xp, tab1, agg1, Wh1, bh1.reshape(1, HID), wij2)
    agg2 = _sc_edge_pass(tab2, eidx, eap, par2, zer)
    x2, tab3 = _tc_layer(HID, x1, tab2, agg2, Wh2, bh2.reshape(1, HID), wij3)
    agg3 = _sc_edge_pass(tab3, eidx, eap, par3, zer)
    out = _tc_final(x2, agg3, Wh3, bh3.reshape(1, HID), batchp,
                    Wl, bl.reshape(1, NG), Wl2, bl2.reshape(1, 1))
    return out
